# baseline (speedup 1.0000x reference)
import jax
import jax.numpy as jnp
from jax import lax
from jax.experimental import pallas as pl
from jax.experimental.pallas import tpu as pltpu
from jax.experimental.pallas import tpu_sc as plsc

_B, _T, _V = 1024, 50, 1000
_VP = 1024
_TP = 56
_NC, _NS = 2, 16
_NW = _NC * _NS
_BPW = _B // _NW
_L = 16

_MESH = plsc.VectorSubcoreMesh(core_axis_name="c", subcore_axis_name="s")


def kernel(idx, table):
    tab_pad = jnp.pad(table, ((0, 0), (0, _VP - _V)))
    idx3 = jnp.pad(idx.reshape(_B, 1, _T), ((0, 0), (0, 0), (0, _TP - _T)))

    @pl.kernel(
        out_type=jax.ShapeDtypeStruct((_B, _T, _V), table.dtype),
        mesh=_MESH,
        scratch_types=[
            pltpu.VMEM((1, _TP), jnp.int32),
            pltpu.VMEM((_TP, _VP), jnp.float32),
            pltpu.VMEM((_T, _V), jnp.float32),
        ],
    )
    def _gather(table_hbm, idx_hbm, out_hbm, idx_v, rows_v, packed_v):
        wid = lax.axis_index("s") * _NC + lax.axis_index("c")
        base = wid * _BPW

        def per_b(j, _):
            b = base + j
            pltpu.sync_copy(idx_hbm.at[b], idx_v)
            pltpu.sync_copy(table_hbm.at[idx_v.at[0]], rows_v)

            def per_row(r, _):
                def per_chunk(k, _):
                    c = k * _L
                    packed_v[r, pl.ds(c, _L)] = rows_v[r, pl.ds(c, _L)]
                    return 0

                lax.fori_loop(0, _V // _L, per_chunk, 0)
                tail = _V - _L
                packed_v[r, pl.ds(tail, _L)] = rows_v[r, pl.ds(tail, _L)]
                return 0

            lax.fori_loop(0, _T, per_row, 0)
            pltpu.sync_copy(packed_v, out_hbm.at[b])
            return 0

        lax.fori_loop(0, _BPW, per_b, 0)

    return _gather(tab_pad, idx3)


# speedup vs baseline: 1.1380x; 1.1380x over previous
import jax
import jax.numpy as jnp
from jax import lax
from jax.experimental import pallas as pl
from jax.experimental.pallas import tpu as pltpu
from jax.experimental.pallas import tpu_sc as plsc

_B, _T, _V = 1024, 50, 1000
_VP = 1024
_TP = 56
_NC, _NS = 2, 16
_NW = _NC * _NS
_BPW = _B // _NW
_L = 16

_MESH = plsc.VectorSubcoreMesh(core_axis_name="c", subcore_axis_name="s")


def kernel(idx, table):
    tab_pad = jnp.pad(table, ((0, 0), (0, _VP - _V)))
    idx3 = jnp.pad(idx.reshape(_B, 1, _T), ((0, 0), (0, 0), (0, _TP - _T)))

    @pl.kernel(
        out_type=jax.ShapeDtypeStruct((_B, _T, _V), table.dtype),
        mesh=_MESH,
        scratch_types=[
            pltpu.VMEM((1, _TP), jnp.int32),
            pltpu.VMEM((_TP, _VP), jnp.float32),
            pltpu.VMEM((_T, _V), jnp.float32),
        ],
    )
    def _gather(table_hbm, idx_hbm, out_hbm, idx_v, rows_v, packed_v):
        wid = lax.axis_index("s") * _NC + lax.axis_index("c")
        base = wid * _BPW

        def per_b(j, _):
            b = base + j
            pltpu.sync_copy(idx_hbm.at[b], idx_v)
            pltpu.sync_copy(table_hbm.at[idx_v.at[0]], rows_v)

            def per_row(r, _):
                for k in range(_V // _L):
                    c = k * _L
                    packed_v[r, pl.ds(c, _L)] = rows_v[r, pl.ds(c, _L)]
                tail = _V - _L
                packed_v[r, pl.ds(tail, _L)] = rows_v[r, pl.ds(tail, _L)]
                return 0

            lax.fori_loop(0, _T, per_row, 0)
            pltpu.sync_copy(packed_v, out_hbm.at[b])
            return 0

        lax.fori_loop(0, _BPW, per_b, 0)

    return _gather(tab_pad, idx3)


# speedup vs baseline: 1.1445x; 1.0057x over previous
import jax
import jax.numpy as jnp
from jax import lax
from jax.experimental import pallas as pl
from jax.experimental.pallas import tpu as pltpu
from jax.experimental.pallas import tpu_sc as plsc

_B, _T, _V = 1024, 50, 1000
_VP = 1024
_TP = 56
_VA = 896
_TA = 48
_NC, _NS = 2, 16
_NW = _NC * _NS
_BPW = _B // _NW
_L = 16

_MESH = plsc.VectorSubcoreMesh(core_axis_name="c", subcore_axis_name="s")


def kernel(idx, table):
    tab_pad = jnp.pad(table, ((0, 0), (0, _VP - _V)))
    idx3 = jnp.pad(idx.reshape(_B, 1, _T), ((0, 0), (0, 0), (0, _TP - _T)))

    @pl.kernel(
        out_type=jax.ShapeDtypeStruct((_B, _T, _V), table.dtype),
        mesh=_MESH,
        scratch_types=[
            pltpu.VMEM((1, _TP), jnp.int32),
            pltpu.VMEM((_TP, _VP), jnp.float32),
            pltpu.VMEM((_TA, _V - _VA), jnp.float32),
            pltpu.VMEM((_T - _TA, _V), jnp.float32),
        ],
    )
    def _gather(table_hbm, idx_hbm, out_hbm, idx_v, rows_v, ctail_v, rtail_v):
        wid = lax.axis_index("s") * _NC + lax.axis_index("c")
        base = wid * _BPW

        def per_b(j, _):
            b = base + j
            pltpu.sync_copy(idx_hbm.at[b], idx_v)
            pltpu.sync_copy(table_hbm.at[idx_v.at[0]], rows_v)

            def col_tail_row(r, _):
                for k in range((_V - _VA) // _L):
                    c = _VA + k * _L
                    ctail_v[r, pl.ds(c - _VA, _L)] = rows_v[r, pl.ds(c, _L)]
                tail = _V - _L
                ctail_v[r, pl.ds(tail - _VA, _L)] = rows_v[r, pl.ds(tail, _L)]
                return 0

            lax.fori_loop(0, _TA, col_tail_row, 0)

            for r in range(_T - _TA):
                for k in range(_V // _L):
                    c = k * _L
                    rtail_v[r, pl.ds(c, _L)] = rows_v[_TA + r, pl.ds(c, _L)]
                tail = _V - _L
                rtail_v[r, pl.ds(tail, _L)] = rows_v[_TA + r, pl.ds(tail, _L)]

            pltpu.sync_copy(
                rows_v.at[pl.ds(0, _TA), pl.ds(0, _VA)],
                out_hbm.at[b, pl.ds(0, _TA), pl.ds(0, _VA)],
            )
            pltpu.sync_copy(ctail_v, out_hbm.at[b, pl.ds(0, _TA), pl.ds(_VA, _V - _VA)])
            pltpu.sync_copy(rtail_v, out_hbm.at[b, pl.ds(_TA, _T - _TA)])
            return 0

        lax.fori_loop(0, _BPW, per_b, 0)

    return _gather(tab_pad, idx3)


# speedup vs baseline: 1.4505x; 1.2674x over previous
import jax
import jax.numpy as jnp
from jax.experimental import pallas as pl
from jax.experimental.pallas import tpu as pltpu
from jax.experimental.pallas import tpu_sc as plsc

_B, _T, _V = 1024, 50, 1000
_VP = 1024
_N = _B * _T
_W = 40
_C = 2

_MESH = plsc.VectorSubcoreMesh(core_axis_name="c", subcore_axis_name="s")


def _gather_chunk(tab_pad, idx3):
    rows = idx3.shape[0] * _W

    @pl.kernel(
        out_type=jax.ShapeDtypeStruct((rows, _VP), tab_pad.dtype),
        mesh=_MESH,
    )
    def _gather(table_hbm, idx_hbm, out_hbm):
        def body(idx_vmem, out_vmem):
            pltpu.sync_copy(table_hbm.at[idx_vmem.at[0, 0]], out_vmem)

        pltpu.emit_pipeline(
            body,
            grid=(rows // _W,),
            in_specs=[pl.BlockSpec((1, 1, _W), lambda i: (i, 0, 0))],
            out_specs=[pl.BlockSpec((_W, _VP), lambda i: (i, 0))],
            core_axis_name=("c", "s"),
            dimension_semantics=(pltpu.PARALLEL,),
        )(idx_hbm, out_hbm)

    return _gather(tab_pad, idx3)


def kernel(idx, table):
    tab_pad = jnp.pad(table, ((0, 0), (0, _VP - _V)))
    idx3 = idx.reshape(_N // _W, 1, _W)
    step = _N // _C // _W
    parts = [
        _gather_chunk(tab_pad, idx3[c * step:(c + 1) * step])[:, :_V]
        for c in range(_C)
    ]
    return jnp.concatenate(parts, axis=0).reshape(_B, _T, _V)
